# baseline (speedup 1.0000x reference)
import math
import functools

import jax
import jax.numpy as jnp
from jax import lax
from jax.experimental import pallas as pl
from jax.experimental.pallas import tpu as pltpu
from jax.experimental.pallas import tpu_sc as plsc

_NT = 1000
_TBL = 1024
_FB = 1024
_COLS = 4 * 64 * 64


def _schedule_tables():
    import numpy as np

    steps = _NT + 1
    xs = np.linspace(0.0, float(_NT), steps, dtype=np.float32)
    acp = np.cos((xs / _NT + 0.008) / (1 + 0.008) * math.pi * 0.5) ** 2
    acp = acp / acp[0]
    betas = np.clip(1.0 - acp[1:] / acp[:-1], 0.0001, 0.9999)
    alphas_cumprod = np.cumprod((1.0 - betas).astype(np.float32))
    c1 = np.sqrt(alphas_cumprod).astype(np.float32)
    c2 = np.sqrt(1.0 - alphas_cumprod).astype(np.float32)
    pad = _TBL - _NT
    return np.pad(c1, (0, pad)), np.pad(c2, (0, pad))


_TBL_NP = None


def _packed_table():
    global _TBL_NP
    if _TBL_NP is None:
        import numpy as np

        c1, c2 = _schedule_tables()
        t = np.zeros((_TBL, _D), np.float32)
        t[:, 0] = c1
        t[:, 1] = c2
        _TBL_NP = jnp.asarray(t)
    return _TBL_NP


_D = 128


def _make_sc_gather(B):
    info = plsc.get_sparse_core_info()
    NC, NS = info.num_cores, info.num_subcores
    NW = NC * NS
    chunk = B // NW
    mesh = plsc.VectorSubcoreMesh(core_axis_name="c", subcore_axis_name="s")

    @functools.partial(
        pl.kernel,
        mesh=mesh,
        out_type=jax.ShapeDtypeStruct((B, _D), jnp.float32),
        scratch_types=[
            pltpu.VMEM((chunk,), jnp.int32),
            pltpu.VMEM((chunk, _D), jnp.float32),
            pltpu.SemaphoreType.DMA,
        ],
    )
    def gather_k(tbl_h, t_h, o_h, idx_v, rows_v, sem):
        wid = lax.axis_index("s") * NC + lax.axis_index("c")
        base = wid * chunk
        pltpu.sync_copy(t_h.at[pl.ds(base, chunk)], idx_v)
        pltpu.async_copy(tbl_h.at[idx_v], rows_v, sem).wait()
        pltpu.sync_copy(rows_v, o_h.at[pl.ds(base, chunk)])

    return gather_k


def _fma_body(c1_ref, c2_ref, x_ref, n_ref, o_ref):
    o_ref[...] = c1_ref[...] * x_ref[...] + c2_ref[...] * n_ref[...]


@jax.jit
def kernel(x, noise, t):
    B = x.shape[0]
    tbl = _packed_table()
    coefs = _make_sc_gather(B)(tbl, t.astype(jnp.int32))
    c1g = coefs[:, 0:1]
    c2g = coefs[:, 1:2]

    C, H, W = x.shape[1], x.shape[2], x.shape[3]
    xT = x.transpose(1, 2, 3, 0).reshape(_COLS, B)
    nT = noise.transpose(1, 2, 3, 0).reshape(_COLS, B)
    c1r = c1g.reshape(1, B)
    c2r = c2g.reshape(1, B)
    out = pl.pallas_call(
        _fma_body,
        grid=(_COLS // _FB,),
        in_specs=[
            pl.BlockSpec((1, B), lambda i: (0, 0)),
            pl.BlockSpec((1, B), lambda i: (0, 0)),
            pl.BlockSpec((_FB, B), lambda i: (i, 0)),
            pl.BlockSpec((_FB, B), lambda i: (i, 0)),
        ],
        out_specs=pl.BlockSpec((_FB, B), lambda i: (i, 0)),
        out_shape=jax.ShapeDtypeStruct((_COLS, B), jnp.float32),
    )(c1r, c2r, xT, nT)
    return out.reshape(C, H, W, B).transpose(3, 0, 1, 2)


# speedup vs baseline: 1.3722x; 1.3722x over previous
import math
import functools

import jax
import jax.numpy as jnp
from jax import lax
from jax.experimental import pallas as pl
from jax.experimental.pallas import tpu as pltpu
from jax.experimental.pallas import tpu_sc as plsc

_NT = 1000
_TBL = 1024
_FB = 1024
_COLS = 4 * 64 * 64


def _schedule_tables():
    import numpy as np

    steps = _NT + 1
    xs = np.linspace(0.0, float(_NT), steps, dtype=np.float32)
    acp = np.cos((xs / _NT + 0.008) / (1 + 0.008) * math.pi * 0.5) ** 2
    acp = acp / acp[0]
    betas = np.clip(1.0 - acp[1:] / acp[:-1], 0.0001, 0.9999)
    alphas_cumprod = np.cumprod((1.0 - betas).astype(np.float32))
    c1 = np.sqrt(alphas_cumprod).astype(np.float32)
    c2 = np.sqrt(1.0 - alphas_cumprod).astype(np.float32)
    pad = _TBL - _NT
    return np.pad(c1, (0, pad)), np.pad(c2, (0, pad))


_TBL_NP = None


def _packed_table():
    global _TBL_NP
    if _TBL_NP is None:
        import numpy as np

        c1, c2 = _schedule_tables()
        t = np.zeros((_TBL, _D), np.float32)
        t[:, 0] = c1
        t[:, 1] = c2
        _TBL_NP = jnp.asarray(t)
    return _TBL_NP


_D = 128


def _make_sc_gather(B):
    info = plsc.get_sparse_core_info()
    NC, NS = info.num_cores, info.num_subcores
    NW = NC * NS
    chunk = B // NW
    mesh = plsc.VectorSubcoreMesh(core_axis_name="c", subcore_axis_name="s")

    @functools.partial(
        pl.kernel,
        mesh=mesh,
        out_type=jax.ShapeDtypeStruct((B, _D), jnp.float32),
        scratch_types=[
            pltpu.VMEM((chunk,), jnp.int32),
            pltpu.VMEM((chunk, _D), jnp.float32),
            pltpu.SemaphoreType.DMA,
        ],
    )
    def gather_k(tbl_h, t_h, o_h, idx_v, rows_v, sem):
        wid = lax.axis_index("s") * NC + lax.axis_index("c")
        base = wid * chunk
        pltpu.sync_copy(t_h.at[pl.ds(base, chunk)], idx_v)
        pltpu.async_copy(tbl_h.at[idx_v], rows_v, sem).wait()
        pltpu.sync_copy(rows_v, o_h.at[pl.ds(base, chunk)])

    return gather_k


def _fma_body(c1_ref, c2_ref, x_ref, n_ref, o_ref):
    o_ref[...] = c1_ref[...] * x_ref[...] + c2_ref[...] * n_ref[...]


def _fused_body(t_ref, tbl_ref, x_ref, n_ref, o_ref, coef_ref):
    @pl.when(pl.program_id(0) == 0)
    def _():
        B = t_ref.shape[1]
        t_row = t_ref[...]
        acc = jnp.zeros((2, B), jnp.float32)
        for kt in range(_TBL // 128):
            k_iota = kt * 128 + jax.lax.broadcasted_iota(jnp.int32, (128, B), 0)
            onehot = (k_iota == t_row).astype(jnp.float32)
            acc = acc + jax.lax.dot_general(
                tbl_ref[:, pl.ds(kt * 128, 128)], onehot,
                (((1,), (0,)), ((), ())),
                preferred_element_type=jnp.float32,
            )
        coef_ref[...] = acc

    c1 = coef_ref[0:1, :]
    c2 = coef_ref[1:2, :]
    o_ref[...] = c1 * x_ref[...] + c2 * n_ref[...]


_TBLT_NP = None


def _packed_tableT():
    global _TBLT_NP
    if _TBLT_NP is None:
        import numpy as np

        c1, c2 = _schedule_tables()
        t = np.zeros((2, _TBL), np.float32)
        t[0, :] = c1
        t[1, :] = c2
        _TBLT_NP = jnp.asarray(t)
    return _TBLT_NP


@jax.jit
def kernel(x, noise, t):
    B = x.shape[0]
    tblT = _packed_tableT()
    t2 = t.astype(jnp.int32).reshape(1, B)

    C, H, W = x.shape[1], x.shape[2], x.shape[3]
    xT = x.transpose(1, 2, 3, 0).reshape(_COLS, B)
    nT = noise.transpose(1, 2, 3, 0).reshape(_COLS, B)
    out = pl.pallas_call(
        _fused_body,
        grid=(_COLS // _FB,),
        in_specs=[
            pl.BlockSpec((1, B), lambda i: (0, 0)),
            pl.BlockSpec((2, _TBL), lambda i: (0, 0)),
            pl.BlockSpec((_FB, B), lambda i: (i, 0)),
            pl.BlockSpec((_FB, B), lambda i: (i, 0)),
        ],
        out_specs=pl.BlockSpec((_FB, B), lambda i: (i, 0)),
        out_shape=jax.ShapeDtypeStruct((_COLS, B), jnp.float32),
        scratch_shapes=[pltpu.VMEM((2, B), jnp.float32)],
    )(t2, tblT, xT, nT)
    return out.reshape(C, H, W, B).transpose(3, 0, 1, 2)
